# Initial kernel scaffold; baseline (speedup 1.0000x reference)
import jax
import jax.numpy as jnp
from jax.experimental import pallas as pl


def kernel(kernel_embedding, x1, x2, w0, b0, w1, b1, w2, b2):
    raise NotImplementedError("write your pallas kernel here")


# speedup vs baseline: 1.9061x; 1.9061x over previous
import jax
import jax.numpy as jnp
from jax import lax
from jax.experimental import pallas as pl
from jax.experimental.pallas import tpu as pltpu


def _round_up(x: int, m: int) -> int:
    return ((x + m - 1) // m) * m


def _cdiv(a: int, b: int) -> int:
    return (a + b - 1) // b


def _gram_kernel(params_ref, x1_ref, x2_ref, o_ref):
    x1b = x1_ref[...].astype(jnp.bfloat16)
    dot = lax.dot_general(
        x1b,
        x2_ref[...],
        dimension_numbers=(((1,), (1,)), ((), ())),
        preferred_element_type=jnp.float32,
    )
    o_ref[...] = params_ref[1] * dot + params_ref[0]


def kernel(kernel_embedding, x1, x2, w0, b0, w1, b1, w2, b2):
    h = jax.nn.relu(kernel_embedding @ w0 + b0)
    h = jax.nn.relu(h @ w1 + b1)
    p = jax.nn.softplus(h @ w2 + b2)
    offset = p[0]
    var2 = p[1] * p[1]
    params = jnp.stack([offset.astype(jnp.float32), var2.astype(jnp.float32)])

    n1, d = x1.shape
    n2, d2 = x2.shape
    assert d == d2

    TM = 512
    n1p = _round_up(n1, TM)
    n2p = _round_up(n2, 128)
    dp = _round_up(d, 256)

    if (n1p, dp) != (n1, d):
        x1 = jnp.pad(x1, ((0, n1p - n1), (0, dp - d)))
    if (n2p, dp) != (n2, d):
        x2 = jnp.pad(x2, ((0, n2p - n2), (0, dp - d)))

    x2b = x2.astype(jnp.bfloat16)

    gm = n1p // TM
    grid = (gm,)

    cost = pl.CostEstimate(
        flops=2 * n1p * n2p * dp,
        transcendentals=0,
        bytes_accessed=n1p * dp * 4 + n2p * dp * 2 + n1p * n2p * 4,
    )

    out = pl.pallas_call(
        _gram_kernel,
        out_shape=jax.ShapeDtypeStruct((n1p, n2p), jnp.float32),
        grid_spec=pltpu.PrefetchScalarGridSpec(
            num_scalar_prefetch=0,
            grid=grid,
            in_specs=[
                pl.BlockSpec(memory_space=pltpu.SMEM),
                pl.BlockSpec((TM, dp), lambda i: (i, 0)),
                pl.BlockSpec((n2p, dp), lambda i: (0, 0)),
            ],
            out_specs=pl.BlockSpec((TM, n2p), lambda i: (i, 0)),
        ),
        compiler_params=pltpu.CompilerParams(
            dimension_semantics=("parallel",),
            vmem_limit_bytes=56 * 1024 * 1024,
        ),
        cost_estimate=cost,
    )(params, x1, x2b)

    if (n1p, n2p) != (n1, n2):
        out = out[:n1, :n2]
    return out


# speedup vs baseline: 2.1944x; 1.1512x over previous
import jax
import jax.numpy as jnp
from jax import lax
from jax.experimental import pallas as pl
from jax.experimental.pallas import tpu as pltpu


def _round_up(x: int, m: int) -> int:
    return ((x + m - 1) // m) * m


def _fused_kernel(emb_ref, w0_ref, b0_ref, w1_ref, b1_ref, w2_ref, b2_ref,
                  x1_ref, x2_ref, o_ref):
    h = jnp.maximum(
        jnp.dot(emb_ref[...], w0_ref[...], preferred_element_type=jnp.float32)
        + b0_ref[...], 0.0)
    h = jnp.maximum(
        jnp.dot(h, w1_ref[...], preferred_element_type=jnp.float32)
        + b1_ref[...], 0.0)
    p = jax.nn.softplus(
        jnp.dot(h, w2_ref[...], preferred_element_type=jnp.float32)
        + b2_ref[...])
    offset = p[0:1, 0:1]
    var2 = p[0:1, 1:2] * p[0:1, 1:2]

    x1b = x1_ref[...].astype(jnp.bfloat16)
    x2b = x2_ref[...].astype(jnp.bfloat16)
    dot = lax.dot_general(
        x1b,
        x2b,
        dimension_numbers=(((1,), (1,)), ((), ())),
        preferred_element_type=jnp.float32,
    )
    o_ref[...] = var2 * dot + offset


def kernel(kernel_embedding, x1, x2, w0, b0, w1, b1, w2, b2):
    n1, d = x1.shape
    n2, d2 = x2.shape
    assert d == d2

    TM = 1024
    n1p = _round_up(n1, TM)
    n2p = _round_up(n2, 128)
    dp = _round_up(d, 256)

    if (n1p, dp) != (n1, d):
        x1 = jnp.pad(x1, ((0, n1p - n1), (0, dp - d)))
    if (n2p, dp) != (n2, d):
        x2 = jnp.pad(x2, ((0, n2p - n2), (0, dp - d)))

    emb2 = kernel_embedding.reshape(1, -1)
    b0r = b0.reshape(1, -1)
    b1r = b1.reshape(1, -1)
    b2r = b2.reshape(1, -1)

    gm = n1p // TM
    grid = (gm,)

    cost = pl.CostEstimate(
        flops=2 * n1p * n2p * dp,
        transcendentals=2,
        bytes_accessed=n1p * dp * 4 + n2p * dp * 4 + n1p * n2p * 4,
    )

    def _const(shape):
        return pl.BlockSpec(shape, lambda i: tuple(0 for _ in shape))

    out = pl.pallas_call(
        _fused_kernel,
        out_shape=jax.ShapeDtypeStruct((n1p, n2p), jnp.float32),
        grid_spec=pltpu.PrefetchScalarGridSpec(
            num_scalar_prefetch=0,
            grid=grid,
            in_specs=[
                _const(emb2.shape),
                _const(w0.shape),
                _const(b0r.shape),
                _const(w1.shape),
                _const(b1r.shape),
                _const(w2.shape),
                _const(b2r.shape),
                pl.BlockSpec((TM, dp), lambda i: (i, 0)),
                pl.BlockSpec((n2p, dp), lambda i: (0, 0)),
            ],
            out_specs=pl.BlockSpec((TM, n2p), lambda i: (i, 0)),
        ),
        compiler_params=pltpu.CompilerParams(
            dimension_semantics=("parallel",),
            vmem_limit_bytes=60 * 1024 * 1024,
        ),
        cost_estimate=cost,
    )(emb2, w0, b0r, w1, b1r, w2, b2r, x1, x2)

    if (n1p, n2p) != (n1, n2):
        out = out[:n1, :n2]
    return out


# speedup vs baseline: 2.2006x; 1.0028x over previous
import jax
import jax.numpy as jnp
from jax import lax
from jax.experimental import pallas as pl
from jax.experimental.pallas import tpu as pltpu


def _round_up(x: int, m: int) -> int:
    return ((x + m - 1) // m) * m


def _fused_kernel(emb_ref, w0_ref, b0_ref, w1_ref, b1_ref, w2_ref, b2_ref,
                  x1_ref, x2_ref, o_ref):
    h = jnp.maximum(
        jnp.dot(emb_ref[...], w0_ref[...], preferred_element_type=jnp.float32)
        + b0_ref[...], 0.0)
    h = jnp.maximum(
        jnp.dot(h, w1_ref[...], preferred_element_type=jnp.float32)
        + b1_ref[...], 0.0)
    p = jax.nn.softplus(
        jnp.dot(h, w2_ref[...], preferred_element_type=jnp.float32)
        + b2_ref[...])
    offset = p[0:1, 0:1]
    var2 = p[0:1, 1:2] * p[0:1, 1:2]

    x1b = (x1_ref[...] * var2).astype(jnp.bfloat16)
    x2b = x2_ref[...].astype(jnp.bfloat16)
    dot = lax.dot_general(
        x1b,
        x2b,
        dimension_numbers=(((1,), (1,)), ((), ())),
        preferred_element_type=jnp.float32,
    )
    o_ref[...] = dot + offset


def kernel(kernel_embedding, x1, x2, w0, b0, w1, b1, w2, b2):
    n1, d = x1.shape
    n2, d2 = x2.shape
    assert d == d2

    TM = 1024
    n1p = _round_up(n1, TM)
    n2p = _round_up(n2, 128)
    dp = _round_up(d, 256)

    if (n1p, dp) != (n1, d):
        x1 = jnp.pad(x1, ((0, n1p - n1), (0, dp - d)))
    if (n2p, dp) != (n2, d):
        x2 = jnp.pad(x2, ((0, n2p - n2), (0, dp - d)))

    emb2 = kernel_embedding.reshape(1, -1)
    b0r = b0.reshape(1, -1)
    b1r = b1.reshape(1, -1)
    b2r = b2.reshape(1, -1)

    gm = n1p // TM
    grid = (gm,)

    cost = pl.CostEstimate(
        flops=2 * n1p * n2p * dp,
        transcendentals=2,
        bytes_accessed=n1p * dp * 4 + n2p * dp * 4 + n1p * n2p * 4,
    )

    def _const(shape):
        return pl.BlockSpec(shape, lambda i: tuple(0 for _ in shape))

    out = pl.pallas_call(
        _fused_kernel,
        out_shape=jax.ShapeDtypeStruct((n1p, n2p), jnp.float32),
        grid_spec=pltpu.PrefetchScalarGridSpec(
            num_scalar_prefetch=0,
            grid=grid,
            in_specs=[
                _const(emb2.shape),
                _const(w0.shape),
                _const(b0r.shape),
                _const(w1.shape),
                _const(b1r.shape),
                _const(w2.shape),
                _const(b2r.shape),
                pl.BlockSpec((TM, dp), lambda i: (i, 0)),
                pl.BlockSpec((n2p, dp), lambda i: (0, 0)),
            ],
            out_specs=pl.BlockSpec((TM, n2p), lambda i: (i, 0)),
        ),
        compiler_params=pltpu.CompilerParams(
            dimension_semantics=("parallel",),
            vmem_limit_bytes=60 * 1024 * 1024,
        ),
        cost_estimate=cost,
    )(emb2, w0, b0r, w1, b1r, w2, b2r, x1, x2)

    if (n1p, n2p) != (n1, n2):
        out = out[:n1, :n2]
    return out
